# Initial kernel scaffold; baseline (speedup 1.0000x reference)
import jax
import jax.numpy as jnp
from jax.experimental import pallas as pl


def kernel(tokens, actions, tok_embed0, tok_embed1, tok_embed2, action_embed, level_embed, pos_embed):
    raise NotImplementedError("write your pallas kernel here")


# speedup vs baseline: 1.4911x; 1.4911x over previous
import functools

import jax
import jax.numpy as jnp
from jax import lax
from jax.experimental import pallas as pl
from jax.experimental.pallas import tpu as pltpu
from jax.experimental.pallas import tpu_sc as plsc

NC = 2
NS = 16
L = 16


def kernel(tokens, actions, tok_embed0, tok_embed1, tok_embed2,
           action_embed, level_embed, pos_embed):
  B, T, _ = tokens.shape
  D = tok_embed0.shape[1]
  NW = NC * NS
  TPB = T // NW
  R = 4 * TPB
  G = 4
  NV = D // L

  idx = jnp.stack(
      [tokens[:, :, 0], tokens[:, :, 1], tokens[:, :, 2], actions], axis=0)
  idx = (idx.reshape(4, B // G, G, NW, TPB)
         .transpose(3, 1, 0, 2, 4)
         .reshape(NW, (B // G) * 4 * G * TPB))

  mesh = plsc.VectorSubcoreMesh(
      core_axis_name="c", subcore_axis_name="s", num_cores=NC,
      num_subcores=NS)

  @functools.partial(
      pl.kernel,
      out_type=jax.ShapeDtypeStruct((B, 4 * T, D), jnp.float32),
      mesh=mesh,
      scratch_types=[
          pltpu.VMEM((4 * G * TPB,), jnp.int32),
          pltpu.VMEM((4, G * TPB, D), jnp.float32),
          pltpu.VMEM((G, R, D), jnp.float32),
          pltpu.VMEM((R, D), jnp.float32),
          pltpu.VMEM((4, D), jnp.float32),
          pltpu.SemaphoreType.DMA,
      ],
  )
  def k(idx_hbm, t0_hbm, t1_hbm, t2_hbm, ta_hbm, lvl_hbm, pos_hbm, out_hbm,
        idx_v, rows_v, outbuf, posbuf, lvlbuf, sem):
    wid = lax.axis_index("s") * NC + lax.axis_index("c")
    t0 = wid * TPB

    pltpu.sync_copy(pos_hbm.at[pl.ds(4 * t0, R)], posbuf)
    pltpu.sync_copy(lvl_hbm, lvlbuf)

    @pl.loop(0, R)
    def poslvl(r):
      j = lax.rem(r, 4)
      for v in range(NV):
        sl = pl.ds(L * v, L)
        posbuf[r, sl] = posbuf[r, sl] + lvlbuf[j, sl]

    tabs = (t0_hbm, t1_hbm, t2_hbm, ta_hbm)

    @pl.loop(0, B // G)
    def group(gi):
      b0 = gi * G
      pltpu.sync_copy(
          idx_hbm.at[wid, pl.ds(gi * (4 * G * TPB), 4 * G * TPB)], idx_v)
      descs = [
          pltpu.async_copy(
              tabs[j].at[idx_v.at[pl.ds(j * G * TPB, G * TPB)]],
              rows_v.at[j], sem)
          for j in range(4)
      ]
      for d in descs:
        d.wait()
      @pl.loop(0, R)
      def per_row(r):
        i = lax.div(r, 4)
        j = lax.rem(r, 4)
        p = [posbuf[r, pl.ds(L * v, L)] for v in range(NV)]

        @pl.loop(0, G)
        def inner(g):
          row = g * TPB + i
          for v in range(NV):
            sl = pl.ds(L * v, L)
            outbuf[g, r, sl] = rows_v[j, row, sl] + p[v]

      pltpu.sync_copy(outbuf, out_hbm.at[pl.ds(b0, G), pl.ds(4 * t0, R)])

  return k(idx, tok_embed0, tok_embed1, tok_embed2, action_embed,
           level_embed, pos_embed)


# speedup vs baseline: 1.8956x; 1.2713x over previous
import functools

import jax
import jax.numpy as jnp
from jax import lax
from jax.experimental import pallas as pl
from jax.experimental.pallas import tpu as pltpu
from jax.experimental.pallas import tpu_sc as plsc

NC = 2
NS = 16
L = 16
NBUF = 3


def kernel(tokens, actions, tok_embed0, tok_embed1, tok_embed2,
           action_embed, level_embed, pos_embed):
  B, T, _ = tokens.shape
  D = tok_embed0.shape[1]
  NW = NC * NS
  TPB = T // NW
  R = 4 * TPB
  G = 2
  NB = B // G
  GI = 4 * G * TPB
  NV = D // L

  idx = jnp.stack(
      [tokens[:, :, 0], tokens[:, :, 1], tokens[:, :, 2], actions], axis=0)
  idx = (idx.reshape(4, NB, G, NW, TPB)
         .transpose(3, 1, 0, 2, 4)
         .reshape(NW, NB * GI))

  mesh = plsc.VectorSubcoreMesh(
      core_axis_name="c", subcore_axis_name="s", num_cores=NC,
      num_subcores=NS)

  @functools.partial(
      pl.kernel,
      out_type=jax.ShapeDtypeStruct((B, T, 4, D), jnp.float32),
      mesh=mesh,
      scratch_types=[
          pltpu.VMEM((NB * GI,), jnp.int32),
          pltpu.VMEM((NBUF, 4, G * TPB, D), jnp.float32),
          pltpu.VMEM((R, D), jnp.float32),
          pltpu.VMEM((4, D), jnp.float32),
          pltpu.SemaphoreType.DMA((NBUF,)),
          pltpu.SemaphoreType.DMA((NBUF,)),
      ],
  )
  def k(idx_hbm, t0_hbm, t1_hbm, t2_hbm, ta_hbm, lvl_hbm, pos_hbm, out_hbm,
        idx_v, rows_v, posbuf, lvlbuf, gsems, wsems):
    wid = lax.axis_index("s") * NC + lax.axis_index("c")
    t0 = wid * TPB
    tabs = (t0_hbm, t1_hbm, t2_hbm, ta_hbm)

    pltpu.sync_copy(idx_hbm.at[wid], idx_v)
    pltpu.sync_copy(pos_hbm.at[pl.ds(4 * t0, R)], posbuf)
    pltpu.sync_copy(lvl_hbm, lvlbuf)

    @pl.loop(0, R)
    def poslvl(r):
      j = lax.rem(r, 4)
      for v in range(NV):
        sl = pl.ds(L * v, L)
        posbuf[r, sl] = posbuf[r, sl] + lvlbuf[j, sl]

    def start_gathers(gi, buf):
      for j in range(4):
        pltpu.async_copy(
            tabs[j].at[idx_v.at[pl.ds(gi * GI + j * (G * TPB), G * TPB)]],
            rows_v.at[buf, j], gsems.at[buf])

    def wait_gathers(buf):
      for j in range(4):
        pltpu.make_async_copy(tabs[j].at[idx_v.at[pl.ds(0, G * TPB)]],
                              rows_v.at[buf, j], gsems.at[buf]).wait()

    def start_writes(gi, buf):
      b0 = gi * G
      for j in range(4):
        for g in range(G):
          pltpu.async_copy(rows_v.at[buf, j, pl.ds(g * TPB, TPB)],
                           out_hbm.at[b0 + g, pl.ds(t0, TPB), j],
                           wsems.at[buf])

    def wait_writes(buf):
      for j in range(4):
        for g in range(G):
          pltpu.make_async_copy(rows_v.at[buf, j, pl.ds(g * TPB, TPB)],
                                out_hbm.at[0, pl.ds(0, TPB), j],
                                wsems.at[buf]).wait()

    start_gathers(0, 0)
    start_gathers(1, 1)

    @pl.loop(0, NB)
    def group(gi):
      buf = lax.rem(gi, NBUF)
      wait_gathers(buf)

      @pl.loop(0, R)
      def per_row(r):
        i = lax.div(r, 4)
        j = lax.rem(r, 4)
        p = [posbuf[r, pl.ds(L * v, L)] for v in range(NV)]
        for g in range(G):
          row = g * TPB + i
          for v in range(NV):
            sl = pl.ds(L * v, L)
            rows_v[buf, j, row, sl] = rows_v[buf, j, row, sl] + p[v]

      start_writes(gi, buf)
      nbuf = lax.rem(gi + 2, NBUF)

      @pl.when(gi + 2 < NB)
      def prefetch():
        @pl.when(gi >= 1)
        def drain():
          wait_writes(nbuf)

        start_gathers(gi + 2, nbuf)

    for b in range(NBUF):
      wait_writes(b)

  out = k(idx, tok_embed0, tok_embed1, tok_embed2, action_embed,
          level_embed, pos_embed)
  return out.reshape(B, 4 * T, D)


# speedup vs baseline: 1.9448x; 1.0259x over previous
import functools

import jax
import jax.numpy as jnp
from jax import lax
from jax.experimental import pallas as pl
from jax.experimental.pallas import tpu as pltpu
from jax.experimental.pallas import tpu_sc as plsc

NC = 2
NS = 16
L = 16


def kernel(tokens, actions, tok_embed0, tok_embed1, tok_embed2,
           action_embed, level_embed, pos_embed):
  B, T, _ = tokens.shape
  D = tok_embed0.shape[1]
  V = tok_embed0.shape[0]
  NW = NC * NS
  TPB = T // NW
  R = 4 * TPB
  G = 2
  NB = B // G
  GI = 4 * G * TPB
  NV = D // L
  VT = 3 * V + action_embed.shape[0]

  table = jnp.concatenate(
      [tok_embed0, tok_embed1, tok_embed2, action_embed], axis=0)
  idx = jnp.stack(
      [tokens[:, :, 0], tokens[:, :, 1] + V, tokens[:, :, 2] + 2 * V,
       actions + 3 * V], axis=0)
  idx = (idx.reshape(4, NB, G, NW, TPB)
         .transpose(3, 1, 0, 2, 4)
         .reshape(NW, NB * GI))

  mesh = plsc.VectorSubcoreMesh(
      core_axis_name="c", subcore_axis_name="s", num_cores=NC,
      num_subcores=NS)

  @functools.partial(
      pl.kernel,
      out_type=jax.ShapeDtypeStruct((B, 4 * T, D), jnp.float32),
      mesh=mesh,
      scratch_types=[
          pltpu.VMEM((NB * GI,), jnp.int32),
          pltpu.VMEM((2, GI, D), jnp.float32),
          pltpu.VMEM((2, G, R, D), jnp.float32),
          pltpu.VMEM((R, D), jnp.float32),
          pltpu.VMEM((4, D), jnp.float32),
          pltpu.SemaphoreType.DMA((2,)),
          pltpu.SemaphoreType.DMA((2,)),
      ],
  )
  def k(tab_hbm, idx_hbm, lvl_hbm, pos_hbm, out_hbm,
        idx_v, rows_v, outbuf, posbuf, lvlbuf, gsems, wsems):
    wid = lax.axis_index("s") * NC + lax.axis_index("c")
    t0 = wid * TPB

    pltpu.sync_copy(idx_hbm.at[wid], idx_v)
    pltpu.sync_copy(pos_hbm.at[pl.ds(4 * t0, R)], posbuf)
    pltpu.sync_copy(lvl_hbm, lvlbuf)

    @pl.loop(0, R)
    def poslvl(r):
      j = lax.rem(r, 4)
      for v in range(NV):
        sl = pl.ds(L * v, L)
        posbuf[r, sl] = posbuf[r, sl] + lvlbuf[j, sl]

    def start_gather(gi, buf):
      pltpu.async_copy(tab_hbm.at[idx_v.at[pl.ds(gi * GI, GI)]],
                       rows_v.at[buf], gsems.at[buf])

    def wait_gather(buf):
      pltpu.make_async_copy(tab_hbm.at[idx_v.at[pl.ds(0, GI)]],
                            rows_v.at[buf], gsems.at[buf]).wait()

    def start_write(gi, buf):
      pltpu.async_copy(outbuf.at[buf],
                       out_hbm.at[pl.ds(gi * G, G), pl.ds(4 * t0, R)],
                       wsems.at[buf])

    def wait_write(buf):
      pltpu.make_async_copy(outbuf.at[buf],
                            out_hbm.at[pl.ds(0, G), pl.ds(0, R)],
                            wsems.at[buf]).wait()

    start_gather(0, 0)
    start_gather(1, 1)

    @pl.loop(0, NB)
    def group(gi):
      buf = lax.rem(gi, 2)
      wait_gather(buf)

      @pl.when(gi >= 2)
      def drain():
        wait_write(buf)

      @pl.loop(0, R)
      def per_row(r):
        i = lax.div(r, 4)
        j = lax.rem(r, 4)
        p = [posbuf[r, pl.ds(L * v, L)] for v in range(NV)]
        for g in range(G):
          srow = j * (G * TPB) + g * TPB + i
          for v in range(NV):
            sl = pl.ds(L * v, L)
            outbuf[buf, g, r, sl] = rows_v[buf, srow, sl] + p[v]

      start_write(gi, buf)

      @pl.when(gi + 2 < NB)
      def prefetch():
        start_gather(gi + 2, buf)

    wait_write(0)
    wait_write(1)

  return k(table, idx, level_embed, pos_embed)


# speedup vs baseline: 3.0028x; 1.5440x over previous
import functools

import jax
import jax.numpy as jnp
from jax import lax
from jax.experimental import pallas as pl
from jax.experimental.pallas import tpu as pltpu
from jax.experimental.pallas import tpu_sc as plsc

NC = 2
NS = 16
L = 16
NBUF = 3


def kernel(tokens, actions, tok_embed0, tok_embed1, tok_embed2,
           action_embed, level_embed, pos_embed):
  B, T, _ = tokens.shape
  D = tok_embed0.shape[1]
  V = tok_embed0.shape[0]
  NW = NC * NS
  TPB = T // NW
  R = 4 * TPB
  G = 2
  NB = B // G
  GI = G * R
  NV = D // L

  table = jnp.concatenate(
      [tok_embed0, tok_embed1, tok_embed2, action_embed], axis=0)
  idx = jnp.stack(
      [tokens[:, :, 0], tokens[:, :, 1] + V, tokens[:, :, 2] + 2 * V,
       actions + 3 * V], axis=0)
  idx = (idx.reshape(4, NB, G, NW, TPB)
         .transpose(3, 1, 2, 4, 0)
         .reshape(NW, NB * GI))

  mesh = plsc.VectorSubcoreMesh(
      core_axis_name="c", subcore_axis_name="s", num_cores=NC,
      num_subcores=NS)

  @functools.partial(
      pl.kernel,
      out_type=jax.ShapeDtypeStruct((B, 4 * T, D), jnp.float32),
      mesh=mesh,
      scratch_types=[
          pltpu.VMEM((NB * GI,), jnp.int32),
          pltpu.VMEM((NBUF, GI, D), jnp.float32),
          pltpu.VMEM((R, D), jnp.float32),
          pltpu.VMEM((4, D), jnp.float32),
          pltpu.SemaphoreType.DMA((NBUF,)),
          pltpu.SemaphoreType.DMA((NBUF,)),
      ],
  )
  def k(tab_hbm, idx_hbm, lvl_hbm, pos_hbm, out_hbm,
        idx_v, rows_v, posbuf, lvlbuf, gsems, wsems):
    wid = lax.axis_index("s") * NC + lax.axis_index("c")
    t0 = wid * TPB

    pltpu.sync_copy(idx_hbm.at[wid], idx_v)
    pltpu.sync_copy(pos_hbm.at[pl.ds(4 * t0, R)], posbuf)
    pltpu.sync_copy(lvl_hbm, lvlbuf)

    @plsc.parallel_loop(0, R, unroll=4)
    def poslvl(r):
      j = lax.rem(r, 4)
      for v in range(NV):
        sl = pl.ds(L * v, L)
        posbuf[r, sl] = posbuf[r, sl] + lvlbuf[j, sl]

    def start_gather(gi, buf):
      pltpu.async_copy(tab_hbm.at[idx_v.at[pl.ds(gi * GI, GI)]],
                       rows_v.at[buf], gsems.at[buf])

    def wait_gather(buf):
      pltpu.make_async_copy(tab_hbm.at[idx_v.at[pl.ds(0, GI)]],
                            rows_v.at[buf], gsems.at[buf]).wait()

    def start_writes(gi, buf):
      for g in range(G):
        pltpu.async_copy(rows_v.at[buf, pl.ds(g * R, R)],
                         out_hbm.at[gi * G + g, pl.ds(4 * t0, R)],
                         wsems.at[buf])

    def wait_writes(buf):
      for g in range(G):
        pltpu.make_async_copy(rows_v.at[buf, pl.ds(g * R, R)],
                              out_hbm.at[0, pl.ds(0, R)],
                              wsems.at[buf]).wait()

    start_gather(0, 0)
    start_gather(1, 1)

    @pl.loop(0, NB)
    def group(gi):
      buf = lax.rem(gi, NBUF)
      wait_gather(buf)

      @plsc.parallel_loop(0, R, unroll=2)
      def per_row(r):
        for v in range(NV):
          sl = pl.ds(L * v, L)
          pv = posbuf[r, sl]
          for g in range(G):
            plsc.addupdate(rows_v.at[buf, g * R + r, sl], pv)

      start_writes(gi, buf)
      nbuf = lax.rem(gi + 2, NBUF)

      @pl.when(gi + 2 < NB)
      def prefetch():
        @pl.when(gi >= 1)
        def drain():
          wait_writes(nbuf)

        start_gather(gi + 2, nbuf)

    for b in range(NBUF):
      wait_writes(b)

  return k(table, idx, level_embed, pos_embed)
